# Initial kernel scaffold; baseline (speedup 1.0000x reference)
import jax
import jax.numpy as jnp
from jax.experimental import pallas as pl


def kernel(feature, patch_lib):
    raise NotImplementedError("write your pallas kernel here")


# speedup vs baseline: 11.4743x; 11.4743x over previous
import functools

import jax
import jax.numpy as jnp
from jax import lax
from jax.experimental import pallas as pl
from jax.experimental.pallas import tpu as pltpu
from jax.experimental.pallas import tpu_sc as plsc

Q, K, D = 784, 100000, 128
KNN = 10
KB = 1024
KPAD = 102400
NBLK = KPAD // KB
NSLOT = 16
QK = Q * KNN
BPAD = 7936
INTMAX = 2**31 - 1


def _b16(v):
    u = lax.bitcast_convert_type(v, jnp.uint32)
    u = u + jnp.uint32(0x7FFF) + ((u >> jnp.uint32(16)) & jnp.uint32(1))
    u = u & jnp.uint32(0xFFFF0000)
    return lax.bitcast_convert_type(u, jnp.float32)


def _argmin_tiebreak(vals, idx_src):
    m = jnp.min(vals, axis=1)
    cand = jnp.where(vals == m[:, None], idx_src, INTMAX)
    return m, jnp.min(cand, axis=1)


def _topk_kernel(f_ref, p_ref, idx_out_ref, bv_ref, bi_ref):
    j = pl.program_id(0)

    @pl.when(j == 0)
    def _():
        bv_ref[...] = jnp.full((Q, NSLOT), jnp.inf, jnp.float32)
        bi_ref[...] = jnp.zeros((Q, NSLOT), jnp.int32)

    f = f_ref[...]
    p = p_ref[...]
    a2 = jnp.sum(f * f, axis=1, keepdims=True)
    b2 = jnp.sum(p * p, axis=1)
    prod = lax.dot_general(f, p, (((1,), (1,)), ((), ())),
                           preferred_element_type=jnp.float32)
    d2 = a2 + b2[None, :] - 2.0 * prod
    dist = jnp.sqrt(jnp.maximum(d2, 0.0))
    col_iota = lax.broadcasted_iota(jnp.int32, (Q, KB), 1)
    gcol = j * KB + col_iota
    dist = jnp.where(gcol < K, dist, jnp.inf)

    rv = bv_ref[...]
    ri = bi_ref[...]
    new_v, new_i = [], []
    for _ in range(KNN):
        bm, bam = _argmin_tiebreak(dist, col_iota)
        rm, rmi = _argmin_tiebreak(rv, ri)
        from_blk = bm < rm
        wv = jnp.where(from_blk, bm, rm)
        wi = jnp.where(from_blk, j * KB + bam, rmi)
        new_v.append(wv[:, None])
        new_i.append(wi[:, None])
        dist = jnp.where(from_blk[:, None] & (col_iota == bam[:, None]),
                         jnp.inf, dist)
        rv = jnp.where((~from_blk)[:, None] & (rv == rm[:, None])
                       & (ri == rmi[:, None]), jnp.inf, rv)
    pad_v = jnp.full((Q, NSLOT - KNN), jnp.inf, jnp.float32)
    pad_i = jnp.zeros((Q, NSLOT - KNN), jnp.int32)
    nv = jnp.concatenate(new_v + [pad_v], axis=1)
    ni = jnp.concatenate(new_i + [pad_i], axis=1)
    bv_ref[...] = nv
    bi_ref[...] = ni

    @pl.when(j == NBLK - 1)
    def _():
        idx_out_ref[...] = ni


def _topk_call(feature, patch_pad):
    return pl.pallas_call(
        _topk_kernel,
        grid=(NBLK,),
        in_specs=[
            pl.BlockSpec((Q, D), lambda j: (0, 0)),
            pl.BlockSpec((KB, D), lambda j: (j, 0)),
        ],
        out_specs=pl.BlockSpec((Q, NSLOT), lambda j: (0, 0)),
        out_shape=jax.ShapeDtypeStruct((Q, NSLOT), jnp.int32),
        scratch_shapes=[
            pltpu.VMEM((Q, NSLOT), jnp.float32),
            pltpu.VMEM((Q, NSLOT), jnp.int32),
        ],
    )(feature, patch_pad)


def _sc_gather_rows(table, idx_flat):
    info = plsc.get_sparse_core_info()
    nw = info.num_cores * info.num_subcores
    bw = BPAD // nw
    mesh = plsc.VectorSubcoreMesh(core_axis_name="c", subcore_axis_name="s")

    @functools.partial(
        pl.kernel, mesh=mesh,
        out_type=jax.ShapeDtypeStruct((BPAD, D), jnp.float32),
        scratch_types=[
            pltpu.VMEM((bw,), jnp.int32),
            pltpu.VMEM((bw, D), jnp.float32),
            pltpu.SemaphoreType.DMA,
        ],
    )
    def _gather(table_hbm, idx_hbm, out_hbm, idx_v, rows_v, sem):
        wid = lax.axis_index("s") * info.num_cores + lax.axis_index("c")
        base = wid * bw
        pltpu.sync_copy(idx_hbm.at[pl.ds(base, bw)], idx_v)
        pltpu.async_copy(table_hbm.at[idx_v], rows_v, sem).wait()
        pltpu.sync_copy(rows_v, out_hbm.at[pl.ds(base, bw)])

    return _gather(table, idx_flat)


def _first_argmax_onehot(c):
    m = jnp.max(c, axis=1)
    iota = lax.broadcasted_iota(jnp.int32, c.shape, 1)
    ji = jnp.min(jnp.where(c == m[:, None], iota, INTMAX), axis=1)
    return (iota == ji[:, None]).astype(jnp.float32)


def _omp_head_kernel(f_ref, kn_ref, dx_ref, g_ref, oh_ref):
    x = f_ref[...]
    xb = _b16(x)
    rb = [_b16(kn_ref[a]) for a in range(KNN)]
    dxs = [jnp.sum(rb[a] * xb, axis=1, keepdims=True) for a in range(KNN)]
    Dx = jnp.concatenate(dxs, axis=1)
    dx_ref[...] = Dx
    g = {}
    for a in range(KNN):
        for b in range(a, KNN):
            g[(a, b)] = jnp.sum(rb[a] * rb[b], axis=1, keepdims=True)
    cols = [g[(a, b)] if a <= b else g[(b, a)]
            for a in range(KNN) for b in range(KNN)]
    g_ref[...] = jnp.concatenate(cols, axis=1)
    oh_ref[...] = _first_argmax_onehot(jnp.abs(Dx))


def _omp_head_call(feature, knn_t):
    return pl.pallas_call(
        _omp_head_kernel,
        in_specs=[
            pl.BlockSpec((Q, D), lambda: (0, 0)),
            pl.BlockSpec((KNN, Q, D), lambda: (0, 0, 0)),
        ],
        out_specs=[
            pl.BlockSpec((Q, KNN), lambda: (0, 0)),
            pl.BlockSpec((Q, KNN * KNN), lambda: (0, 0)),
            pl.BlockSpec((Q, KNN), lambda: (0, 0)),
        ],
        out_shape=[
            jax.ShapeDtypeStruct((Q, KNN), jnp.float32),
            jax.ShapeDtypeStruct((Q, KNN * KNN), jnp.float32),
            jax.ShapeDtypeStruct((Q, KNN), jnp.float32),
        ],
    )(feature, knn_t)


def _resid_matvec(x, rb, coef):
    acc = jnp.zeros(x.shape, jnp.float32)
    cb = _b16(coef)
    for a in range(KNN):
        acc = acc + cb[:, a:a + 1] * rb[a]
    return x - acc


def _omp_select_kernel(f_ref, kn_ref, coef_ref, mask_ref, oh_ref):
    x = f_ref[...]
    rb = [_b16(kn_ref[a]) for a in range(KNN)]
    r = _resid_matvec(x, rb, coef_ref[...])
    rbv = _b16(r)
    corrs = [jnp.sum(rb[a] * rbv, axis=1, keepdims=True) for a in range(KNN)]
    corr = jnp.abs(jnp.concatenate(corrs, axis=1))
    c = jnp.where(mask_ref[...] > 0, -jnp.inf, corr)
    oh_ref[...] = _first_argmax_onehot(c)


def _omp_select_call(feature, knn_t, coef, mask):
    return pl.pallas_call(
        _omp_select_kernel,
        in_specs=[
            pl.BlockSpec((Q, D), lambda: (0, 0)),
            pl.BlockSpec((KNN, Q, D), lambda: (0, 0, 0)),
            pl.BlockSpec((Q, KNN), lambda: (0, 0)),
            pl.BlockSpec((Q, KNN), lambda: (0, 0)),
        ],
        out_specs=pl.BlockSpec((Q, KNN), lambda: (0, 0)),
        out_shape=jax.ShapeDtypeStruct((Q, KNN), jnp.float32),
    )(feature, knn_t, coef, mask)


def _omp_final_kernel(f_ref, kn_ref, coef_ref, nn_ref, dict_ref, s_ref):
    x = f_ref[...]
    nn_ref[...] = kn_ref[0]
    rb = [_b16(kn_ref[a]) for a in range(KNN)]
    cb = _b16(coef_ref[...])
    acc = jnp.zeros(x.shape, jnp.float32)
    for a in range(KNN):
        acc = acc + cb[:, a:a + 1] * rb[a]
    dict_ref[...] = acc
    diff = x - acc + 1e-12
    min_val = jnp.sqrt(jnp.sum(diff * diff, axis=1, keepdims=True))
    s_ref[...] = jnp.max(min_val, axis=0, keepdims=True)


def _omp_final_call(feature, knn_t, coef):
    return pl.pallas_call(
        _omp_final_kernel,
        in_specs=[
            pl.BlockSpec((Q, D), lambda: (0, 0)),
            pl.BlockSpec((KNN, Q, D), lambda: (0, 0, 0)),
            pl.BlockSpec((Q, KNN), lambda: (0, 0)),
        ],
        out_specs=[
            pl.BlockSpec((Q, D), lambda: (0, 0)),
            pl.BlockSpec((Q, D), lambda: (0, 0)),
            pl.BlockSpec((1, 1), lambda: (0, 0)),
        ],
        out_shape=[
            jax.ShapeDtypeStruct((Q, D), jnp.float32),
            jax.ShapeDtypeStruct((Q, D), jnp.float32),
            jax.ShapeDtypeStruct((1, 1), jnp.float32),
        ],
    )(feature, knn_t, coef)


def _masked_solve(G, mask, Dx):
    def one(Gq, mq, dq):
        mo = mq[:, None] * mq[None, :]
        A = Gq * mo + jnp.diag(1.0 - mq)
        return jnp.linalg.solve(A, mq * dq)
    return jax.vmap(one)(G, mask, Dx)


def kernel(feature, patch_lib):
    patch_pad = jnp.concatenate(
        [patch_lib, jnp.zeros((KPAD - K, D), jnp.float32)], axis=0)
    knn16 = _topk_call(feature, patch_pad)
    knn_idx = knn16[:, :KNN]

    flat = jnp.transpose(knn_idx).reshape(-1)
    flat = jnp.concatenate(
        [flat, jnp.zeros((BPAD - QK,), jnp.int32)])
    rows = _sc_gather_rows(patch_lib, flat)
    knn_t = rows[:QK].reshape(KNN, Q, D)

    Dx, G100, oh1 = _omp_head_call(feature, knn_t)
    G = G100.reshape(Q, KNN, KNN)
    mask = oh1
    coef = _masked_solve(G, mask, Dx)
    oh2 = _omp_select_call(feature, knn_t, coef, mask)
    mask = mask + oh2
    coef = _masked_solve(G, mask, Dx)
    oh3 = _omp_select_call(feature, knn_t, coef, mask)
    mask = mask + oh3
    coef = _masked_solve(G, mask, Dx)

    nn, dict_f, s = _omp_final_call(feature, knn_t, coef)
    return (nn, dict_f, knn_idx, jnp.reshape(s, ()))


# speedup vs baseline: 15.3723x; 1.3397x over previous
import functools

import jax
import jax.numpy as jnp
from jax import lax
from jax.experimental import pallas as pl
from jax.experimental.pallas import tpu as pltpu
from jax.experimental.pallas import tpu_sc as plsc

Q, K, D = 784, 100000, 128
KNN = 10
KB = 1024
KPAD = 102400
NBLK = KPAD // KB
NSLOT = 16
QK = Q * KNN
BPAD = 7936
INTMAX = 2**31 - 1


def _b16(v):
    u = lax.bitcast_convert_type(v, jnp.uint32)
    u = u + jnp.uint32(0x7FFF) + ((u >> jnp.uint32(16)) & jnp.uint32(1))
    u = u & jnp.uint32(0xFFFF0000)
    return lax.bitcast_convert_type(u, jnp.float32)


def _argmin_tiebreak(vals, idx_src):
    m = jnp.min(vals, axis=1)
    cand = jnp.where(vals == m[:, None], idx_src, INTMAX)
    return m, jnp.min(cand, axis=1)


def _topk_kernel(f_ref, p_ref, idx_out_ref, bv_ref, bi_ref):
    j = pl.program_id(0)

    @pl.when(j == 0)
    def _():
        bv_ref[...] = jnp.full((Q, NSLOT), jnp.inf, jnp.float32)
        bi_ref[...] = jnp.zeros((Q, NSLOT), jnp.int32)

    f = f_ref[...]
    p = p_ref[...]
    a2 = jnp.sum(f * f, axis=1, keepdims=True)
    b2 = jnp.sum(p * p, axis=1)
    prod = lax.dot_general(f, p, (((1,), (1,)), ((), ())),
                           preferred_element_type=jnp.float32)
    d2 = a2 + b2[None, :] - 2.0 * prod
    dist = jnp.sqrt(jnp.maximum(d2, 0.0))
    col_iota = lax.broadcasted_iota(jnp.int32, (Q, KB), 1)
    gcol = j * KB + col_iota
    dist = jnp.where(gcol < K, dist, jnp.inf)

    rv = bv_ref[...]
    ri = bi_ref[...]
    new_v, new_i = [], []
    for _ in range(KNN):
        bm, bam = _argmin_tiebreak(dist, col_iota)
        rm, rmi = _argmin_tiebreak(rv, ri)
        from_blk = bm < rm
        wv = jnp.where(from_blk, bm, rm)
        wi = jnp.where(from_blk, j * KB + bam, rmi)
        new_v.append(wv[:, None])
        new_i.append(wi[:, None])
        dist = jnp.where(from_blk[:, None] & (col_iota == bam[:, None]),
                         jnp.inf, dist)
        rv = jnp.where((~from_blk)[:, None] & (rv == rm[:, None])
                       & (ri == rmi[:, None]), jnp.inf, rv)
    pad_v = jnp.full((Q, NSLOT - KNN), jnp.inf, jnp.float32)
    pad_i = jnp.zeros((Q, NSLOT - KNN), jnp.int32)
    nv = jnp.concatenate(new_v + [pad_v], axis=1)
    ni = jnp.concatenate(new_i + [pad_i], axis=1)
    bv_ref[...] = nv
    bi_ref[...] = ni

    @pl.when(j == NBLK - 1)
    def _():
        idx_out_ref[...] = ni


def _topk_call(feature, patch_pad):
    return pl.pallas_call(
        _topk_kernel,
        grid=(NBLK,),
        in_specs=[
            pl.BlockSpec((Q, D), lambda j: (0, 0)),
            pl.BlockSpec((KB, D), lambda j: (j, 0)),
        ],
        out_specs=pl.BlockSpec((Q, NSLOT), lambda j: (0, 0)),
        out_shape=jax.ShapeDtypeStruct((Q, NSLOT), jnp.int32),
        scratch_shapes=[
            pltpu.VMEM((Q, NSLOT), jnp.float32),
            pltpu.VMEM((Q, NSLOT), jnp.int32),
        ],
    )(feature, patch_pad)


def _sc_gather_rows(table, idx_flat):
    info = plsc.get_sparse_core_info()
    nw = info.num_cores * info.num_subcores
    bw = BPAD // nw
    mesh = plsc.VectorSubcoreMesh(core_axis_name="c", subcore_axis_name="s")

    @functools.partial(
        pl.kernel, mesh=mesh,
        out_type=jax.ShapeDtypeStruct((BPAD, D), jnp.float32),
        scratch_types=[
            pltpu.VMEM((bw,), jnp.int32),
            pltpu.VMEM((bw, D), jnp.float32),
            pltpu.SemaphoreType.DMA,
        ],
    )
    def _gather(table_hbm, idx_hbm, out_hbm, idx_v, rows_v, sem):
        wid = lax.axis_index("s") * info.num_cores + lax.axis_index("c")
        base = wid * bw
        pltpu.sync_copy(idx_hbm.at[pl.ds(base, bw)], idx_v)
        pltpu.async_copy(table_hbm.at[idx_v], rows_v, sem).wait()
        pltpu.sync_copy(rows_v, out_hbm.at[pl.ds(base, bw)])

    return _gather(table, idx_flat)


def _first_argmax_onehot(c):
    m = jnp.max(c, axis=1)
    iota = lax.broadcasted_iota(jnp.int32, c.shape, 1)
    ji = jnp.min(jnp.where(c == m[:, None], iota, INTMAX), axis=1)
    return (iota == ji[:, None]).astype(jnp.float32)


def _omp_head_kernel(f_ref, kn_ref, dx_ref, g_ref, oh_ref):
    x = f_ref[...]
    xb = _b16(x)
    rb = [_b16(kn_ref[a]) for a in range(KNN)]
    dxs = [jnp.sum(rb[a] * xb, axis=1, keepdims=True) for a in range(KNN)]
    Dx = jnp.concatenate(dxs, axis=1)
    dx_ref[...] = Dx
    g = {}
    for a in range(KNN):
        for b in range(a, KNN):
            g[(a, b)] = jnp.sum(rb[a] * rb[b], axis=1, keepdims=True)
    cols = [g[(a, b)] if a <= b else g[(b, a)]
            for a in range(KNN) for b in range(KNN)]
    g_ref[...] = jnp.concatenate(cols, axis=1)
    oh_ref[...] = _first_argmax_onehot(jnp.abs(Dx))


def _omp_head_call(feature, knn_t):
    return pl.pallas_call(
        _omp_head_kernel,
        in_specs=[
            pl.BlockSpec((Q, D), lambda: (0, 0)),
            pl.BlockSpec((KNN, Q, D), lambda: (0, 0, 0)),
        ],
        out_specs=[
            pl.BlockSpec((Q, KNN), lambda: (0, 0)),
            pl.BlockSpec((Q, KNN * KNN), lambda: (0, 0)),
            pl.BlockSpec((Q, KNN), lambda: (0, 0)),
        ],
        out_shape=[
            jax.ShapeDtypeStruct((Q, KNN), jnp.float32),
            jax.ShapeDtypeStruct((Q, KNN * KNN), jnp.float32),
            jax.ShapeDtypeStruct((Q, KNN), jnp.float32),
        ],
    )(feature, knn_t)


def _omp_select_kernel(f_ref, kn_ref, coef_ref, mask_ref, oh_ref):
    x = f_ref[...]
    rb = [_b16(kn_ref[a]) for a in range(KNN)]
    cb = _b16(coef_ref[...])
    acc = jnp.zeros(x.shape, jnp.float32)
    for a in range(KNN):
        acc = acc + cb[:, a:a + 1] * rb[a]
    r = x - acc
    rbv = _b16(r)
    corrs = [jnp.sum(rb[a] * rbv, axis=1, keepdims=True) for a in range(KNN)]
    corr = jnp.abs(jnp.concatenate(corrs, axis=1))
    c = jnp.where(mask_ref[...] > 0, -jnp.inf, corr)
    oh_ref[...] = _first_argmax_onehot(c)


def _omp_select_call(feature, knn_t, coef, mask):
    return pl.pallas_call(
        _omp_select_kernel,
        in_specs=[
            pl.BlockSpec((Q, D), lambda: (0, 0)),
            pl.BlockSpec((KNN, Q, D), lambda: (0, 0, 0)),
            pl.BlockSpec((Q, KNN), lambda: (0, 0)),
            pl.BlockSpec((Q, KNN), lambda: (0, 0)),
        ],
        out_specs=pl.BlockSpec((Q, KNN), lambda: (0, 0)),
        out_shape=jax.ShapeDtypeStruct((Q, KNN), jnp.float32),
    )(feature, knn_t, coef, mask)


def _omp_final_kernel(f_ref, kn_ref, dx_ref, g_ref, oh1_ref, oh2_ref, oh3_ref,
                      nn_ref, dict_ref, s_ref):
    x = f_ref[...]
    nn_ref[...] = kn_ref[0]
    rows = [kn_ref[a] for a in range(KNN)]
    Dx = dx_ref[...]
    ohs = [oh1_ref[...], oh2_ref[...], oh3_ref[...]]

    sels, dxs, grows = [], [], []
    for oh in ohs:
        sel = jnp.zeros((Q, D), jnp.float32)
        for p in range(KNN):
            sel = sel + oh[:, p:p + 1] * rows[p]
        sels.append(sel)
        dxs.append(jnp.sum(Dx * oh, axis=1))
        grow = jnp.zeros((Q, KNN), jnp.float32)
        for p in range(KNN):
            grow = grow + oh[:, p:p + 1] * g_ref[:, p * KNN:(p + 1) * KNN]
        grows.append(grow)

    def gsel(a, b):
        return jnp.sum(grows[a] * ohs[b], axis=1)

    g11, g12, g13 = gsel(0, 0), gsel(0, 1), gsel(0, 2)
    g22, g23, g33 = gsel(1, 1), gsel(1, 2), gsel(2, 2)
    dx1, dx2, dx3 = dxs
    m11 = g22 * g33 - g23 * g23
    m12 = g13 * g23 - g12 * g33
    m13 = g12 * g23 - g13 * g22
    m22 = g11 * g33 - g13 * g13
    m23 = g13 * g12 - g11 * g23
    m33 = g11 * g22 - g12 * g12
    det3 = g11 * m11 + g12 * m12 + g13 * m13
    c1f = (m11 * dx1 + m12 * dx2 + m13 * dx3) / det3
    c2f = (m12 * dx1 + m22 * dx2 + m23 * dx3) / det3
    c3f = (m13 * dx1 + m23 * dx2 + m33 * dx3) / det3

    acc = (_b16(c1f)[:, None] * _b16(sels[0])
           + _b16(c2f)[:, None] * _b16(sels[1])
           + _b16(c3f)[:, None] * _b16(sels[2]))
    dict_ref[...] = acc
    diff = x - acc + 1e-12
    min_val = jnp.sqrt(jnp.sum(diff * diff, axis=1, keepdims=True))
    s_ref[...] = jnp.max(min_val, axis=0, keepdims=True)


def _omp_final_call(feature, knn_t, Dx, G100, oh1, oh2, oh3):
    return pl.pallas_call(
        _omp_final_kernel,
        in_specs=[
            pl.BlockSpec((Q, D), lambda: (0, 0)),
            pl.BlockSpec((KNN, Q, D), lambda: (0, 0, 0)),
            pl.BlockSpec((Q, KNN), lambda: (0, 0)),
            pl.BlockSpec((Q, KNN * KNN), lambda: (0, 0)),
            pl.BlockSpec((Q, KNN), lambda: (0, 0)),
            pl.BlockSpec((Q, KNN), lambda: (0, 0)),
            pl.BlockSpec((Q, KNN), lambda: (0, 0)),
        ],
        out_specs=[
            pl.BlockSpec((Q, D), lambda: (0, 0)),
            pl.BlockSpec((Q, D), lambda: (0, 0)),
            pl.BlockSpec((1, 1), lambda: (0, 0)),
        ],
        out_shape=[
            jax.ShapeDtypeStruct((Q, D), jnp.float32),
            jax.ShapeDtypeStruct((Q, D), jnp.float32),
            jax.ShapeDtypeStruct((1, 1), jnp.float32),
        ],
    )(feature, knn_t, Dx, G100, oh1, oh2, oh3)


def _masked_solve(G, mask, Dx):
    def one(Gq, mq, dq):
        mo = mq[:, None] * mq[None, :]
        A = Gq * mo + jnp.diag(1.0 - mq)
        return jnp.linalg.solve(A, mq * dq)
    return jax.vmap(one)(G, mask, Dx)


def kernel(feature, patch_lib):
    patch_pad = jnp.concatenate(
        [patch_lib, jnp.zeros((KPAD - K, D), jnp.float32)], axis=0)
    knn16 = _topk_call(feature, patch_pad)
    knn_idx = knn16[:, :KNN]

    flat = jnp.transpose(knn_idx).reshape(-1)
    flat = jnp.concatenate(
        [flat, jnp.zeros((BPAD - QK,), jnp.int32)])
    rows = _sc_gather_rows(patch_lib, flat)
    knn_t = rows[:QK].reshape(KNN, Q, D)

    Dx, G100, oh1 = _omp_head_call(feature, knn_t)
    G = G100.reshape(Q, KNN, KNN)
    mask = oh1
    coef = _masked_solve(G, mask, Dx)
    oh2 = _omp_select_call(feature, knn_t, coef, mask)
    mask = mask + oh2
    coef = _masked_solve(G, mask, Dx)
    oh3 = _omp_select_call(feature, knn_t, coef, mask)

    nn, dict_f, s = _omp_final_call(feature, knn_t, Dx, G100, oh1, oh2, oh3)
    return (nn, dict_f, knn_idx, jnp.reshape(s, ()))
